# baseline (speedup 1.0000x reference)
import functools

import jax
import jax.numpy as jnp
from jax import lax
from jax.experimental import pallas as pl
from jax.experimental.pallas import tpu as pltpu
from jax.experimental.pallas import tpu_sc as plsc

_THRESH = 0.7
_THRESH_BITS = 0x3F333333
_MIN_KEPT = 100000
_NW = 32
_NB = 4096


def _fused_body(pred_ref, tgt_ref, stats_ref, pbits_ref, *rest, emit_nll):
    x = pred_ref[0]
    t = tgt_ref[0]
    thr = jnp.float32(_THRESH)
    m = jnp.max(x, axis=0)
    e = jnp.exp(x - m[None])
    s = jnp.sum(e, axis=0)
    cls = lax.broadcasted_iota(jnp.int32, x.shape, 0)
    onehot = cls == t[None]
    tl = jnp.sum(jnp.where(onehot, x, 0.0), axis=0)
    el = jnp.sum(jnp.where(onehot, e, 0.0), axis=0)
    p = el / s
    nll = m + jnp.log(s) - tl
    lt = p < thr
    tot = jnp.sum(jnp.where(lt, nll, 0.0))
    c_lt = jnp.sum(lt.astype(jnp.float32))

    first = (pl.program_id(0) == 0) & (pl.program_id(1) == 0)

    @pl.when(first)
    def _():
        stats_ref[0, 0] = tot
        stats_ref[0, 1] = c_lt

    @pl.when(jnp.logical_not(first))
    def _():
        stats_ref[0, 0] += tot
        stats_ref[0, 1] += c_lt

    pbits_ref[0] = lax.bitcast_convert_type(p, jnp.int32)
    if emit_nll:
        rest[0][0] = nll


def _run_fused(predict, target, emit_nll, bh):
    n, c, h, w = predict.shape
    grid = (n, h // bh)
    in_specs = [
        pl.BlockSpec((1, c, bh, w), lambda i, j: (i, 0, j, 0)),
        pl.BlockSpec((1, bh, w), lambda i, j: (i, j, 0)),
    ]
    out_shapes = [
        jax.ShapeDtypeStruct((1, 2), jnp.float32),
        jax.ShapeDtypeStruct((n, h, w), jnp.int32),
    ]
    out_specs = [
        pl.BlockSpec((1, 2), lambda i, j: (0, 0), memory_space=pltpu.SMEM),
        pl.BlockSpec((1, bh, w), lambda i, j: (i, j, 0)),
    ]
    if emit_nll:
        out_shapes.append(jax.ShapeDtypeStruct((n, h, w), jnp.float32))
        out_specs.append(pl.BlockSpec((1, bh, w), lambda i, j: (i, j, 0)))
    return pl.pallas_call(
        functools.partial(_fused_body, emit_nll=emit_nll),
        grid=grid,
        in_specs=in_specs,
        out_specs=out_specs,
        out_shape=out_shapes,
    )(predict, target)


def _sc_hist(numel, key_shift, key_mask, sel_shift):
    per = numel // _NW
    ch = per // 16
    masked = sel_shift is not None
    scratch = [
        pltpu.VMEM((per,), jnp.int32),
        pltpu.VMEM((_NB,), jnp.int32),
    ]
    if masked:
        scratch.append(pltpu.VMEM((16,), jnp.int32))
    mesh = plsc.VectorSubcoreMesh(core_axis_name="c", subcore_axis_name="s")

    def body(bits_hbm, *rest):
        if masked:
            sel_hbm, out_hbm, data, hist, selv = rest
        else:
            out_hbm, data, hist = rest
        wid = lax.axis_index("s") * 2 + lax.axis_index("c")
        pltpu.sync_copy(bits_hbm.at[pl.ds(wid * per, per)], data)
        if masked:
            pltpu.sync_copy(sel_hbm, selv)
            sel = selv[...]

        def zero(i, _):
            hist[pl.ds(i * 16, 16)] = jnp.zeros((16,), jnp.int32)
            return 0

        lax.fori_loop(0, _NB // 16, zero, 0)

        ones = jnp.ones((16,), jnp.int32)

        def sbody(i, _):
            x = data[pl.ds(i * 16, 16)]
            key = lax.shift_right_logical(x, key_shift) if key_shift else x
            key = lax.bitwise_and(key, key_mask)
            if masked:
                msk = lax.shift_right_logical(x, sel_shift) == sel
                plsc.addupdate_scatter(hist, [key], ones, mask=msk)
            else:
                plsc.addupdate_scatter(hist, [key], ones)
            return 0

        lax.fori_loop(0, ch, sbody, 0)
        pltpu.sync_copy(hist, out_hbm.at[wid])

    return pl.kernel(
        body,
        out_type=jax.ShapeDtypeStruct((_NW, _NB), jnp.int32),
        mesh=mesh,
        scratch_types=scratch,
        compiler_params=pltpu.CompilerParams(needs_layout_passes=False),
    )


def _finder_body(k_ref, h_ref, out_ref):
    kf = k_ref[0, 0].astype(jnp.float32)
    h = h_ref[...].astype(jnp.float32)
    g2 = jnp.sum(h, axis=0)
    rowsum = jnp.sum(g2, axis=1, keepdims=True)
    ri = lax.broadcasted_iota(jnp.int32, (32, 32), 0)
    ci = lax.broadcasted_iota(jnp.int32, (32, 32), 1)
    mtri = (ci < ri).astype(jnp.float32)
    cer = jnp.dot(mtri, rowsum, preferred_element_type=jnp.float32,
                  precision=lax.Precision.HIGHEST)
    a = lax.broadcasted_iota(jnp.int32, (128, 128), 0)
    b = lax.broadcasted_iota(jnp.int32, (128, 128), 1)
    ntri = (a <= b).astype(jnp.float32)
    cir = jnp.dot(g2, ntri, preferred_element_type=jnp.float32,
                  precision=lax.Precision.HIGHEST)
    cum = cer + cir
    le = cum <= kf
    out_ref[0, 0] = jnp.sum(le.astype(jnp.int32))
    out_ref[0, 1] = jnp.max(jnp.where(le, cum, 0.0)).astype(jnp.int32)


def _finder(hist, k):
    return pl.pallas_call(
        _finder_body,
        in_specs=[
            pl.BlockSpec(memory_space=pltpu.SMEM),
            pl.BlockSpec(),
        ],
        out_specs=pl.BlockSpec(memory_space=pltpu.SMEM),
        out_shape=jax.ShapeDtypeStruct((1, 2), jnp.int32),
    )(k.reshape(1, 1), hist.reshape(_NW, 32, 128))


def _sc_select(bits_flat, kept_idx):
    numel = bits_flat.shape[0]
    k32 = jnp.int32(kept_idx)
    h1 = _sc_hist(numel, 18, _NB - 1, None)(bits_flat)
    f1 = _finder(h1, k32)
    b1, cb1 = f1[0, 0], f1[0, 1]

    def refine(_):
        h2 = _sc_hist(numel, 6, _NB - 1, 18)(
            bits_flat, jnp.full((16,), b1, jnp.int32))
        f2 = _finder(h2, k32 - cb1)
        b2, cb2 = f2[0, 0], f2[0, 1]
        b12 = b1 * _NB + b2
        h3 = _sc_hist(numel, 0, 63, 6)(
            bits_flat, jnp.full((16,), b12, jnp.int32))
        f3 = _finder(h3, k32 - cb1 - cb2)
        return b12 * 64 + f3[0, 0]

    return lax.cond(b1 >= (_THRESH_BITS >> 18), refine,
                    lambda _: jnp.int32(0), None)


def _reduce_body(thr_ref, pbits_ref, nll_ref, out_ref):
    thr = thr_ref[0, 0]
    lt = pbits_ref[0] < thr
    tot = jnp.sum(jnp.where(lt, nll_ref[0], 0.0))
    cnt = jnp.sum(lt.astype(jnp.float32))
    first = (pl.program_id(0) == 0) & (pl.program_id(1) == 0)

    @pl.when(first)
    def _():
        out_ref[0, 0] = tot
        out_ref[0, 1] = cnt

    @pl.when(jnp.logical_not(first))
    def _():
        out_ref[0, 0] += tot
        out_ref[0, 1] += cnt


def _masked_reduce(pbits, nll, thr_bits, bh):
    n, h, w = pbits.shape
    grid = (n, h // bh)
    out = pl.pallas_call(
        _reduce_body,
        grid=grid,
        in_specs=[
            pl.BlockSpec(memory_space=pltpu.SMEM),
            pl.BlockSpec((1, bh, w), lambda i, j: (i, j, 0)),
            pl.BlockSpec((1, bh, w), lambda i, j: (i, j, 0)),
        ],
        out_specs=pl.BlockSpec((1, 2), lambda i, j: (0, 0), memory_space=pltpu.SMEM),
        out_shape=jax.ShapeDtypeStruct((1, 2), jnp.float32),
    )(thr_bits.reshape(1, 1), pbits, nll)
    return out[0, 0], out[0, 1]


def _final(total, count):
    return jnp.where(count > 0, total / jnp.maximum(count, 1.0), total)


def kernel(predict, target):
    n, c, h, w = predict.shape
    numel = n * h * w
    kept_idx = max(min(_MIN_KEPT * n, numel - 1), 0)
    bh = 64 if h % 64 == 0 else 16

    stats, pbits = _run_fused(predict, target, emit_nll=False, bh=bh)
    v_bits = _sc_select(pbits.reshape(-1), kept_idx)

    def common(_):
        return _final(stats[0, 0], stats[0, 1])

    def rare(_):
        _, pbits2, nll = _run_fused(predict, target, emit_nll=True, bh=bh)
        total, count = _masked_reduce(pbits2, nll, v_bits, bh)
        return _final(total, count)

    return lax.cond(v_bits <= jnp.int32(_THRESH_BITS), common, rare,
                    operand=None)
